# baseline (speedup 1.0000x reference)
import functools

import jax
import jax.numpy as jnp
from jax import lax
from jax.experimental import pallas as pl
from jax.experimental.pallas import tpu as pltpu
from jax.experimental.pallas import tpu_sc as plsc

N_NODES = 100000
D = 128
CH = 160
NCHUNK = -(-N_NODES // CH)
LAST = NCHUNK - 1
LAST_ROW0 = N_NODES - CH
NC = 2
NS = 16
NW = NC * NS
BASEN = NCHUNK // NW
EXTRA = NCHUNK % NW
STEPS = BASEN + 1
STAGE = STEPS * CH
NBUF = 5
GLEAD = 2


def _sc_gather(table, idx):
    mesh = plsc.VectorSubcoreMesh(core_axis_name="c", subcore_axis_name="s")

    @functools.partial(
        pl.kernel,
        out_type=jax.ShapeDtypeStruct((N_NODES, D), jnp.float32),
        mesh=mesh,
        scratch_types=(
            [pltpu.VMEM((STAGE,), jnp.int32),
             pltpu.VMEM_SHARED((1024, D), jnp.float32)]
            + [pltpu.VMEM((CH, D), jnp.float32) for _ in range(NBUF)]
            + [pltpu.SemaphoreType.DMA for _ in range(2 * NBUF)]
        ),
    )
    def k(table_hbm, idx_hbm, out_hbm, idx_v, table_sp, *rest):
        bufs = rest[:NBUF]
        gsem = rest[NBUF:2 * NBUF]
        wsem = rest[2 * NBUF:]

        wid = lax.axis_index("s") * NC + lax.axis_index("c")
        start_chunk = BASEN * wid + jnp.minimum(wid, EXTRA)
        n_last = jnp.where(wid < EXTRA, BASEN, BASEN - 1)
        base_stage = jnp.where(wid == NW - 1, N_NODES - STAGE,
                               start_chunk * CH)
        base_stage = pl.multiple_of(base_stage, 8)

        def out_row(t):
            c = start_chunk + jnp.minimum(t, n_last)
            return pl.multiple_of(jnp.where(c == LAST, LAST_ROW0, c * CH), 8)

        def gather_desc(t, slot):
            io = pl.multiple_of(out_row(t) - base_stage, 8)
            return pltpu.make_async_copy(
                table_sp.at[idx_v.at[pl.ds(io, CH)]],
                bufs[slot], gsem[slot])

        def write_desc(t, slot):
            return pltpu.make_async_copy(
                bufs[slot], out_hbm.at[pl.ds(out_row(t), CH)],
                wsem[slot])

        sid = lax.axis_index("s")
        rows_per_tile = 1024 // NS
        pltpu.sync_copy(table_hbm.at[pl.ds(sid * rows_per_tile, rows_per_tile)],
                        table_sp.at[pl.ds(sid * rows_per_tile, rows_per_tile)])
        pltpu.sync_copy(idx_hbm.at[pl.ds(base_stage, STAGE)], idx_v)
        plsc.subcore_barrier()

        MID_LO = NBUF - GLEAD
        MID_HI = STEPS - GLEAD

        def step(t, tmod, wait_write, start_gather):
            if wait_write:
                write_desc(t - NBUF + GLEAD, (tmod + GLEAD) % NBUF).wait()
            if start_gather:
                gather_desc(t + GLEAD, (tmod + GLEAD) % NBUF).start()
            gather_desc(t, tmod).wait()
            write_desc(t, tmod).start()

        for t in range(GLEAD):
            gather_desc(t, t % NBUF).start()
        for t in range(MID_LO):
            step(t, t % NBUF, False, True)

        def body(g, carry):
            for b in range(NBUF):
                t = MID_LO + g * NBUF + b
                step(t, (MID_LO + b) % NBUF, True, True)
            return carry

        ngroups = (MID_HI - MID_LO) // NBUF
        lax.fori_loop(0, ngroups, body, 0)
        for t in range(MID_LO + ngroups * NBUF, MID_HI):
            step(t, t % NBUF, True, True)
        for t in range(MID_HI, STEPS):
            step(t, t % NBUF, False, False)
        for t in range(STEPS - NBUF, STEPS):
            write_desc(t, t % NBUF).wait()

    return k(table, idx)


def kernel(graph_feat, node_segment):
    idx = node_segment.astype(jnp.int32)
    return _sc_gather(graph_feat, idx)


# speedup vs baseline: 1.0125x; 1.0125x over previous
import functools

import jax
import jax.numpy as jnp
from jax import lax
from jax.experimental import pallas as pl
from jax.experimental.pallas import tpu as pltpu
from jax.experimental.pallas import tpu_sc as plsc

N_NODES = 100000
D = 128
CH = 128
NCHUNK = -(-N_NODES // CH)
LAST = NCHUNK - 1
LAST_ROW0 = N_NODES - CH
NC = 2
NS = 16
NW = NC * NS
BASEN = NCHUNK // NW
EXTRA = NCHUNK % NW
STEPS = BASEN + 1
STAGE = STEPS * CH
NBUF = 7
GLEAD = 2


def _sc_gather(table, idx):
    mesh = plsc.VectorSubcoreMesh(core_axis_name="c", subcore_axis_name="s")

    @functools.partial(
        pl.kernel,
        out_type=jax.ShapeDtypeStruct((N_NODES, D), jnp.float32),
        mesh=mesh,
        scratch_types=(
            [pltpu.VMEM((STAGE,), jnp.int32),
             pltpu.VMEM_SHARED((1024, D), jnp.float32)]
            + [pltpu.VMEM((CH, D), jnp.float32) for _ in range(NBUF)]
            + [pltpu.SemaphoreType.DMA for _ in range(2 * NBUF)]
        ),
    )
    def k(table_hbm, idx_hbm, out_hbm, idx_v, table_sp, *rest):
        bufs = rest[:NBUF]
        gsem = rest[NBUF:2 * NBUF]
        wsem = rest[2 * NBUF:]

        wid = lax.axis_index("s") * NC + lax.axis_index("c")
        start_chunk = BASEN * wid + jnp.minimum(wid, EXTRA)
        n_last = jnp.where(wid < EXTRA, BASEN, BASEN - 1)
        base_stage = jnp.where(wid == NW - 1, N_NODES - STAGE,
                               start_chunk * CH)
        base_stage = pl.multiple_of(base_stage, 8)

        def out_row(t):
            c = start_chunk + jnp.minimum(t, n_last)
            return pl.multiple_of(jnp.where(c == LAST, LAST_ROW0, c * CH), 8)

        def gather_desc(t, slot):
            io = pl.multiple_of(out_row(t) - base_stage, 8)
            return pltpu.make_async_copy(
                table_sp.at[idx_v.at[pl.ds(io, CH)]],
                bufs[slot], gsem[slot])

        def write_desc(t, slot):
            return pltpu.make_async_copy(
                bufs[slot], out_hbm.at[pl.ds(out_row(t), CH)],
                wsem[slot])

        sid = lax.axis_index("s")
        rows_per_tile = 1024 // NS
        pltpu.sync_copy(table_hbm.at[pl.ds(sid * rows_per_tile, rows_per_tile)],
                        table_sp.at[pl.ds(sid * rows_per_tile, rows_per_tile)])
        pltpu.sync_copy(idx_hbm.at[pl.ds(base_stage, STAGE)], idx_v)
        plsc.subcore_barrier()

        MID_LO = NBUF - GLEAD
        MID_HI = STEPS - GLEAD

        def step(t, tmod, wait_write, start_gather):
            if wait_write:
                write_desc(t - NBUF + GLEAD, (tmod + GLEAD) % NBUF).wait()
            if start_gather:
                gather_desc(t + GLEAD, (tmod + GLEAD) % NBUF).start()
            gather_desc(t, tmod).wait()
            write_desc(t, tmod).start()

        for t in range(GLEAD):
            gather_desc(t, t % NBUF).start()
        for t in range(MID_LO):
            step(t, t % NBUF, False, True)

        def body(g, carry):
            for b in range(NBUF):
                t = MID_LO + g * NBUF + b
                step(t, (MID_LO + b) % NBUF, True, True)
            return carry

        ngroups = (MID_HI - MID_LO) // NBUF
        lax.fori_loop(0, ngroups, body, 0)
        for t in range(MID_LO + ngroups * NBUF, MID_HI):
            step(t, t % NBUF, True, True)
        for t in range(MID_HI, STEPS):
            step(t, t % NBUF, False, False)
        for t in range(STEPS - NBUF, STEPS):
            write_desc(t, t % NBUF).wait()

    return k(table, idx)


def kernel(graph_feat, node_segment):
    idx = node_segment.astype(jnp.int32)
    return _sc_gather(graph_feat, idx)
